# Initial kernel scaffold; baseline (speedup 1.0000x reference)
import jax
import jax.numpy as jnp
from jax.experimental import pallas as pl


def kernel(edge_index, edge_logits, edge_feats, node_feats, W_edge, b_edge, W_ih, W_hh, b_ih, b_hh):
    raise NotImplementedError("write your pallas kernel here")


# speedup vs baseline: 8.0731x; 8.0731x over previous
import jax
import jax.numpy as jnp
from jax import lax
from jax.experimental import pallas as pl
from jax.experimental.pallas import tpu as pltpu
from jax.experimental.pallas import tpu_sc as plsc

N = 10000
E = 320000
D = 128

NC = 2
NS = 16
L = 16
NW = NC * NS
EPT = E // NW
K = 80
NCHUNK = EPT // K
SROWS = 80
STRIPE = 624
TAIL = N - NS * STRIPE
ZR = 208


def _sc_body(dst_hbm, logit_hbm, feats_hbm, p_hbm, s_hbm,
             idx_v, logit_v, feats_v, s2_v, zero_v, p_sh):
    cid = lax.axis_index("c")
    sid = lax.axis_index("s")
    wid = cid * NS + sid

    def zz(i, _):
        for j in range(D // L):
            zero_v[i, pl.ds(j * L, L)] = jnp.zeros((L,), jnp.float32)
        return 0
    lax.fori_loop(0, ZR, zz, 0)

    def zs(i, _):
        for j in range(D // L):
            s2_v[i, pl.ds(j * L, L)] = jnp.zeros((L,), jnp.float32)
        return 0
    lax.fori_loop(0, SROWS, zs, 0)

    row0 = sid * STRIPE
    for z in range(STRIPE // ZR):
        pltpu.sync_copy(zero_v, p_sh.at[pl.ds(row0 + z * ZR, ZR)])

    @pl.when(sid == NS - 1)
    def _():
        pltpu.sync_copy(zero_v.at[pl.ds(0, TAIL)],
                        p_sh.at[pl.ds(NS * STRIPE, TAIL)])
    plsc.subcore_barrier()

    base0 = wid * EPT
    lane0 = lax.iota(jnp.int32, L) == 0

    def chunk(ci, _):
        base = base0 + ci * K
        pltpu.sync_copy(dst_hbm.at[pl.ds(base, K)], idx_v)
        pltpu.sync_copy(logit_hbm.at[pl.ds(base, K)], logit_v)
        pltpu.sync_copy(feats_hbm.at[pl.ds(base, K)], feats_v)

        def edge(e, _):
            e16 = jnp.full((L,), e, jnp.int32)
            w = jnp.exp(plsc.load_gather(logit_v, [e16]))
            d16 = plsc.load_gather(idx_v, [e16])
            r16 = lax.shift_right_logical(d16, 7)
            c16 = lax.bitwise_and(d16, 127)
            plsc.addupdate_scatter(s2_v, [r16, c16], w, mask=lane0)
            for j in range(D // L):
                sl = pl.ds(j * L, L)
                feats_v[e, sl] = feats_v[e, sl] * w
            return 0
        lax.fori_loop(0, K, edge, 0)

        pltpu.sync_copy(feats_v, p_sh.at[idx_v], add=True)
        return 0
    lax.fori_loop(0, NCHUNK, chunk, 0)

    pltpu.sync_copy(s2_v, s_hbm.at[wid])
    plsc.subcore_barrier()
    pltpu.sync_copy(p_sh.at[pl.ds(row0, STRIPE)],
                    p_hbm.at[cid, pl.ds(row0, STRIPE)])

    @pl.when(sid == NS - 1)
    def _():
        pltpu.sync_copy(p_sh.at[pl.ds(NS * STRIPE, TAIL)],
                        p_hbm.at[cid, pl.ds(NS * STRIPE, TAIL)])


@jax.jit
def _sc_accum(dst, logits, edge_feats):
    mesh = plsc.VectorSubcoreMesh(core_axis_name="c", subcore_axis_name="s")
    return pl.kernel(
        _sc_body,
        out_type=(jax.ShapeDtypeStruct((NC, N, D), jnp.float32),
                  jax.ShapeDtypeStruct((NW, SROWS, D), jnp.float32)),
        mesh=mesh,
        scratch_types=[
            pltpu.VMEM((K,), jnp.int32),
            pltpu.VMEM((K,), jnp.float32),
            pltpu.VMEM((K, D), jnp.float32),
            pltpu.VMEM((SROWS, D), jnp.float32),
            pltpu.VMEM((ZR, D), jnp.float32),
            pltpu.VMEM_SHARED((N, D), jnp.float32),
        ],
        compiler_params=pltpu.CompilerParams(needs_layout_passes=False),
    )(dst, logits, edge_feats)


def _tc_body(p_ref, st_ref, nf_ref, we_ref, be_ref, wih_ref, whh_ref,
             bih_ref, bhh_ref, out_ref):
    psum = p_ref[0] + p_ref[1]
    ssum = jnp.sum(st_ref[...], axis=1, keepdims=True)
    nonempty = ssum > 0.0
    inv = jnp.where(nonempty, 1.0 / jnp.where(nonempty, ssum, 1.0), 0.0)
    a = psum * inv
    et = lax.dot_general(a, we_ref[...], (((1,), (1,)), ((), ())),
                         preferred_element_type=jnp.float32) + be_ref[...]
    c = jnp.where(nonempty, et, 0.0)
    ctx = jnp.where(c > 0.0, c, jnp.exp(jnp.minimum(c, 0.0)) - 1.0)
    nf = nf_ref[...]
    gi = lax.dot_general(ctx, wih_ref[...], (((1,), (1,)), ((), ())),
                         preferred_element_type=jnp.float32) + bih_ref[...]
    gh = lax.dot_general(nf, whh_ref[...], (((1,), (1,)), ((), ())),
                         preferred_element_type=jnp.float32) + bhh_ref[...]
    r = jax.nn.sigmoid(gi[:, :D] + gh[:, :D])
    z = jax.nn.sigmoid(gi[:, D:2 * D] + gh[:, D:2 * D])
    n = jnp.tanh(gi[:, 2 * D:] + r * gh[:, 2 * D:])
    h = (1.0 - z) * n + z * nf
    out_ref[...] = jnp.maximum(h, 0.0)


@jax.jit
def _tc_dense(p, s_t, node_feats, W_edge, b_edge, W_ih, W_hh, b_ih, b_hh):
    full = lambda shape: pl.BlockSpec(shape, lambda: (0,) * len(shape))
    return pl.pallas_call(
        _tc_body,
        in_specs=[
            full((NC, N, D)),
            full((N, NW)),
            full((N, D)),
            full((D, D)),
            full((1, D)),
            full((3 * D, D)),
            full((3 * D, D)),
            full((1, 3 * D)),
            full((1, 3 * D)),
        ],
        out_specs=full((N, D)),
        out_shape=jax.ShapeDtypeStruct((N, D), jnp.float32),
    )(p, s_t, node_feats, W_edge, b_edge, W_ih, W_hh, b_ih, b_hh)


def kernel(edge_index, edge_logits, edge_feats, node_feats,
           W_edge, b_edge, W_ih, W_hh, b_ih, b_hh):
    dst = edge_index[1].astype(jnp.int32)
    logits = edge_logits[:, 0].astype(jnp.float32)
    p, s = _sc_accum(dst, logits, edge_feats)
    s_t = s.reshape(NW, SROWS * D)[:, :N].T
    return _tc_dense(p, s_t, node_feats, W_edge,
                     b_edge.reshape(1, D), W_ih, W_hh,
                     b_ih.reshape(1, 3 * D), b_hh.reshape(1, 3 * D))


# speedup vs baseline: 15.4948x; 1.9193x over previous
import jax
import jax.numpy as jnp
from jax import lax
from jax.experimental import pallas as pl
from jax.experimental.pallas import tpu as pltpu
from jax.experimental.pallas import tpu_sc as plsc

N = 10000
E = 320000
D = 128

NC = 2
NS = 16
L = 16
NW = NC * NS
EPT = E // NW
K = 40
NCHUNK = EPT // K
NBUF = 5
SROWS = 80
STRIPE = 624
TAIL = N - NS * STRIPE
ZR = 48


def _sc_body(dst3_hbm, logit3_hbm, feats_hbm, p_hbm, s_hbm,
             idx_v, logit_v, feats_v, s2_v, zero_v, p_sh,
             in_sems, sc_sems):
    cid = lax.axis_index("c")
    sid = lax.axis_index("s")
    wid = cid * NS + sid
    base0 = wid * EPT

    def fire_in(ci, b):
        pltpu.async_copy(feats_hbm.at[pl.ds(base0 + ci * K, K)],
                         feats_v.at[b], in_sems.at[b])
        pltpu.async_copy(dst3_hbm.at[wid, ci], idx_v.at[b], in_sems.at[b])
        pltpu.async_copy(logit3_hbm.at[wid, ci], logit_v.at[b], in_sems.at[b])

    def wait_in(ci, b):
        pltpu.make_async_copy(feats_hbm.at[pl.ds(base0 + ci * K, K)],
                              feats_v.at[b], in_sems.at[b]).wait()
        pltpu.make_async_copy(dst3_hbm.at[wid, ci], idx_v.at[b],
                              in_sems.at[b]).wait()
        pltpu.make_async_copy(logit3_hbm.at[wid, ci], logit_v.at[b],
                              in_sems.at[b]).wait()

    for b in range(NBUF):
        fire_in(b, b)

    def zz(i, _):
        for j in range(D // L):
            zero_v[i, pl.ds(j * L, L)] = jnp.zeros((L,), jnp.float32)
        return 0
    lax.fori_loop(0, ZR, zz, 0)

    def zs(i, _):
        for j in range(D // L):
            s2_v[i, pl.ds(j * L, L)] = jnp.zeros((L,), jnp.float32)
        return 0
    lax.fori_loop(0, SROWS, zs, 0)

    row0 = sid * STRIPE
    for z in range(STRIPE // ZR):
        pltpu.sync_copy(zero_v, p_sh.at[pl.ds(row0 + z * ZR, ZR)])

    @pl.when(sid == NS - 1)
    def _():
        pltpu.sync_copy(zero_v.at[pl.ds(0, TAIL)],
                        p_sh.at[pl.ds(NS * STRIPE, TAIL)])

    plsc.subcore_barrier()

    lane0 = lax.iota(jnp.int32, L) == 0

    def fire_sc(b):
        pltpu.async_copy(feats_v.at[b], p_sh.at[idx_v.at[b]],
                         sc_sems.at[b], add=True)

    def drain_sc(b):
        pltpu.make_async_copy(feats_v.at[b], p_sh.at[idx_v.at[b]],
                              sc_sems.at[b]).wait()

    def outer(g, _):
        for b in range(NBUF):
            ci = g * NBUF + b
            wait_in(ci, b)

            b16 = jnp.full((L,), b, jnp.int32)

            def edge(e, _):
                e16 = jnp.full((L,), e, jnp.int32)
                w = jnp.exp(plsc.load_gather(logit_v, [b16, e16]))
                d16 = plsc.load_gather(idx_v, [b16, e16])
                r16 = lax.shift_right_logical(d16, 7)
                c16 = lax.bitwise_and(d16, 127)
                plsc.addupdate_scatter(s2_v, [r16, c16], w, mask=lane0)
                for j in range(D // L):
                    sl = pl.ds(j * L, L)
                    feats_v[b, e, sl] = feats_v[b, e, sl] * w
                return 0
            lax.fori_loop(0, K, edge, 0, unroll=4)
            fire_sc(b)

            pb = (b - 3) % NBUF

            @pl.when(ci >= 3)
            def _():
                drain_sc(pb)

            @pl.when(jnp.logical_and(ci >= 3, ci <= NCHUNK - 3))
            def _():
                fire_in(ci + 2, pb)
        return 0
    lax.fori_loop(0, NCHUNK // NBUF, outer, 0)

    for b in range(NBUF - 3, NBUF):
        drain_sc(b)

    pltpu.sync_copy(s2_v, s_hbm.at[wid])
    plsc.subcore_barrier()
    pltpu.sync_copy(p_sh.at[pl.ds(row0, STRIPE)],
                    p_hbm.at[cid, pl.ds(row0, STRIPE)])

    @pl.when(sid == NS - 1)
    def _():
        pltpu.sync_copy(p_sh.at[pl.ds(NS * STRIPE, TAIL)],
                        p_hbm.at[cid, pl.ds(NS * STRIPE, TAIL)])


@jax.jit
def _sc_accum(dst3, logit3, edge_feats):
    mesh = plsc.VectorSubcoreMesh(core_axis_name="c", subcore_axis_name="s")
    return pl.kernel(
        _sc_body,
        out_type=(jax.ShapeDtypeStruct((NC, N, D), jnp.float32),
                  jax.ShapeDtypeStruct((NW, SROWS, D), jnp.float32)),
        mesh=mesh,
        scratch_types=[
            pltpu.VMEM((NBUF, K), jnp.int32),
            pltpu.VMEM((NBUF, K), jnp.float32),
            pltpu.VMEM((NBUF, K, D), jnp.float32),
            pltpu.VMEM((SROWS, D), jnp.float32),
            pltpu.VMEM((ZR, D), jnp.float32),
            pltpu.VMEM_SHARED((N, D), jnp.float32),
            pltpu.SemaphoreType.DMA((NBUF,)),
            pltpu.SemaphoreType.DMA((NBUF,)),
        ],
        compiler_params=pltpu.CompilerParams(needs_layout_passes=False),
    )(dst3, logit3, edge_feats)


def _tc_body(p_ref, st_ref, nf_ref, we_ref, be_ref, wih_ref, whh_ref,
             bih_ref, bhh_ref, out_ref):
    psum = p_ref[0] + p_ref[1]
    ssum = jnp.sum(st_ref[...], axis=1, keepdims=True)
    nonempty = ssum > 0.0
    inv = jnp.where(nonempty, 1.0 / jnp.where(nonempty, ssum, 1.0), 0.0)
    a = psum * inv
    et = lax.dot_general(a, we_ref[...], (((1,), (1,)), ((), ())),
                         preferred_element_type=jnp.float32) + be_ref[...]
    c = jnp.where(nonempty, et, 0.0)
    ctx = jnp.where(c > 0.0, c, jnp.exp(jnp.minimum(c, 0.0)) - 1.0)
    nf = nf_ref[...]
    gi = lax.dot_general(ctx, wih_ref[...], (((1,), (1,)), ((), ())),
                         preferred_element_type=jnp.float32) + bih_ref[...]
    gh = lax.dot_general(nf, whh_ref[...], (((1,), (1,)), ((), ())),
                         preferred_element_type=jnp.float32) + bhh_ref[...]
    r = jax.nn.sigmoid(gi[:, :D] + gh[:, :D])
    z = jax.nn.sigmoid(gi[:, D:2 * D] + gh[:, D:2 * D])
    n = jnp.tanh(gi[:, 2 * D:] + r * gh[:, 2 * D:])
    h = (1.0 - z) * n + z * nf
    out_ref[...] = jnp.maximum(h, 0.0)


@jax.jit
def _tc_dense(p, s_t, node_feats, W_edge, b_edge, W_ih, W_hh, b_ih, b_hh):
    full = lambda shape: pl.BlockSpec(shape, lambda: (0,) * len(shape))
    return pl.pallas_call(
        _tc_body,
        in_specs=[
            full((NC, N, D)),
            full((N, NW)),
            full((N, D)),
            full((D, D)),
            full((1, D)),
            full((3 * D, D)),
            full((3 * D, D)),
            full((1, 3 * D)),
            full((1, 3 * D)),
        ],
        out_specs=full((N, D)),
        out_shape=jax.ShapeDtypeStruct((N, D), jnp.float32),
    )(p, s_t, node_feats, W_edge, b_edge, W_ih, W_hh, b_ih, b_hh)


def kernel(edge_index, edge_logits, edge_feats, node_feats,
           W_edge, b_edge, W_ih, W_hh, b_ih, b_hh):
    dst3 = edge_index[1].astype(jnp.int32).reshape(NW, NCHUNK, K)
    logit3 = edge_logits[:, 0].astype(jnp.float32).reshape(NW, NCHUNK, K)
    p, s = _sc_accum(dst3, logit3, edge_feats)
    s_t = s.reshape(NW, SROWS * D)[:, :N].T
    return _tc_dense(p, s_t, node_feats, W_edge,
                     b_edge.reshape(1, D), W_ih, W_hh,
                     b_ih.reshape(1, 3 * D), b_hh.reshape(1, 3 * D))
